# baseline (speedup 1.0000x reference)
import functools

import jax
import jax.numpy as jnp
from jax import lax
from jax.experimental import pallas as pl
from jax.experimental.pallas import tpu as pltpu
from jax.experimental.pallas import tpu_sc as plsc

N, E, H, G, L = 10000, 320000, 128, 64, 3
NC, NS = 2, 16
NW = NC * NS
EPW = E // NW
CH = 125
NCHUNK = EPW // CH
SB = 10
NSUP = NCHUNK // SB
NP2 = 10240
RPT = NP2 // NS
ZCH = 32
NPAD = 10240
DSL = NPAD // NS
BR = 1000
NBLK = N // BR


@functools.partial(
    pl.kernel,
    out_type=jax.ShapeDtypeStruct((NC, NPAD, 16), jnp.float32),
    mesh=plsc.VectorSubcoreMesh(core_axis_name="c", subcore_axis_name="s"),
    scratch_types=[
        pltpu.VMEM((NCHUNK, CH), jnp.int32),
        pltpu.VMEM((CH, 16), jnp.float32),
        pltpu.VMEM((128, 16), jnp.float32),
        pltpu.VMEM_SHARED((NPAD, 16), jnp.float32),
    ],
)
def _deg_kernel(dst_hbm, degp_hbm, idv, ones_v, zb, acc):
    cid = lax.axis_index("c")
    sid = lax.axis_index("s")
    wid = sid * NC + cid
    zeros16 = jnp.zeros((16,), jnp.float32)
    ones16 = jnp.ones((16,), jnp.float32)

    def fill(r, c):
        zb[r, :] = zeros16
        return c

    lax.fori_loop(0, 128, fill, 0)

    def fill1(r, c):
        ones_v[r, :] = ones16
        return c

    lax.fori_loop(0, CH, fill1, 0)
    for t in range(DSL // 128):
        pltpu.sync_copy(zb, acc.at[pl.ds(sid * DSL + t * 128, 128)])
    pltpu.sync_copy(dst_hbm.at[wid], idv)
    plsc.subcore_barrier()

    def cb(ch, c):
        pltpu.sync_copy(ones_v, acc.at[idv.at[ch]], add=True)
        return c

    lax.fori_loop(0, NCHUNK, cb, 0)
    plsc.subcore_barrier()
    pltpu.sync_copy(acc.at[pl.ds(sid * DSL, DSL)],
                    degp_hbm.at[cid, pl.ds(sid * DSL, DSL)])


@functools.partial(
    pl.kernel,
    out_type=jax.ShapeDtypeStruct((NC, NP2, H), jnp.float32),
    mesh=plsc.VectorSubcoreMesh(core_axis_name="c", subcore_axis_name="s"),
    scratch_types=[
        pltpu.VMEM((2, SB, CH), jnp.int32),
        pltpu.VMEM((CH, H), jnp.float32),
        pltpu.VMEM((CH, H), jnp.float32),
        pltpu.VMEM((ZCH, H), jnp.float32),
        pltpu.VMEM_SHARED((NP2, H), jnp.float32),
        pltpu.SemaphoreType.DMA,
        pltpu.SemaphoreType.DMA,
    ],
)
def _msg_kernel(hs_hbm, idx_hbm, p_hbm, ibuf, buf0, buf1, zb, acc, sem0,
                sem1):
    cid = lax.axis_index("c")
    sid = lax.axis_index("s")
    wid = sid * NC + cid
    zeros16 = jnp.zeros((16,), jnp.float32)

    @pl.when(cid == 0)
    def _():
        pltpu.sync_copy(hs_hbm.at[pl.ds(sid * RPT, RPT)],
                        acc.at[pl.ds(sid * RPT, RPT)])

    @pl.when(cid == 1)
    def _():
        def z1(r, c):
            for cc in range(H // 16):
                zb[r, pl.ds(cc * 16, 16)] = zeros16
            return c

        lax.fori_loop(0, ZCH, z1, 0)
        for t in range(RPT // ZCH):
            pltpu.sync_copy(zb, acc.at[pl.ds(sid * RPT + t * ZCH, ZCH)])

    plsc.subcore_barrier()

    bufs = (buf0, buf1)
    sems = (sem0, sem1)
    pltpu.sync_copy(idx_hbm.at[wid, 0], ibuf)
    pltpu.async_copy(hs_hbm.at[ibuf.at[0, 0]], buf0, sem0)

    def sup(s, c):
        for j in range(SB):
            cur, csem = bufs[j % 2], sems[j % 2]
            if j < SB - 1:
                pltpu.async_copy(hs_hbm.at[ibuf.at[0, j + 1]],
                                 bufs[(j + 1) % 2], sems[(j + 1) % 2])
            pltpu.make_async_copy(hs_hbm.at[ibuf.at[0, j]], cur, csem).wait()
            pltpu.sync_copy(cur, acc.at[ibuf.at[1, j]], add=True)

        @pl.when(s < NSUP - 1)
        def _():
            pltpu.sync_copy(idx_hbm.at[wid, s + 1], ibuf)
            pltpu.async_copy(hs_hbm.at[ibuf.at[0, 0]], buf0, sem0)

        return c

    lax.fori_loop(0, NSUP, sup, 0)
    plsc.subcore_barrier()
    pltpu.sync_copy(acc.at[pl.ds(sid * RPT, RPT)],
                    p_hbm.at[cid, pl.ds(sid * RPT, RPT)])


def _k1_body(x_ref, we_ref, be_ref, d0_ref, d1_ref, w0_ref, hs_ref, dinv_ref):
    deg = d0_ref[...] + d1_ref[...] + 1.0
    dinv = lax.rsqrt(deg)
    h0 = jnp.dot(x_ref[...], we_ref[...], preferred_element_type=jnp.float32)
    h0 = jnp.maximum(h0 + be_ref[...], 0.0)
    hs = jnp.dot(h0, w0_ref[...], preferred_element_type=jnp.float32) * dinv
    hs_ref[...] = hs
    dinv_ref[...] = dinv


_k1 = pl.pallas_call(
    _k1_body,
    grid=(NBLK,),
    in_specs=[
        pl.BlockSpec((BR, H), lambda i: (i, 0)),
        pl.BlockSpec((H, H), lambda i: (0, 0)),
        pl.BlockSpec((1, H), lambda i: (0, 0)),
        pl.BlockSpec((BR, 1), lambda i: (i, 0)),
        pl.BlockSpec((BR, 1), lambda i: (i, 0)),
        pl.BlockSpec((H, H), lambda i: (0, 0)),
    ],
    out_specs=[
        pl.BlockSpec((BR, H), lambda i: (i, 0)),
        pl.BlockSpec((BR, 1), lambda i: (i, 0)),
    ],
    out_shape=[
        jax.ShapeDtypeStruct((NP2, H), jnp.float32),
        jax.ShapeDtypeStruct((N, 1), jnp.float32),
    ],
)


def _layer_body(p_ref, dinv_ref, bnm, bnv, bng, bnb, cb, wn_ref,
                out_ref):
    dinv = dinv_ref[...]
    t = dinv * (p_ref[0] + p_ref[1]) + cb[...]
    t = (t - bnm[...]) * lax.rsqrt(bnv[...] + 1e-5) * bng[...] + bnb[...]
    h = jnp.maximum(t, 0.0)
    out_ref[...] = jnp.dot(h, wn_ref[...],
                           preferred_element_type=jnp.float32) * dinv


_layer = pl.pallas_call(
    _layer_body,
    grid=(NBLK,),
    in_specs=[
        pl.BlockSpec((NC, BR, H), lambda i: (0, i, 0)),
        pl.BlockSpec((BR, 1), lambda i: (i, 0)),
        pl.BlockSpec((1, H), lambda i: (0, 0)),
        pl.BlockSpec((1, H), lambda i: (0, 0)),
        pl.BlockSpec((1, H), lambda i: (0, 0)),
        pl.BlockSpec((1, H), lambda i: (0, 0)),
        pl.BlockSpec((1, H), lambda i: (0, 0)),
        pl.BlockSpec((H, H), lambda i: (0, 0)),
    ],
    out_specs=pl.BlockSpec((BR, H), lambda i: (i, 0)),
    out_shape=jax.ShapeDtypeStruct((NP2, H), jnp.float32),
)


def _k4_body(p_ref, dinv_ref, bnm, bnv, bng, bnb, cb, brow_ref,
             bcol_ref, w1, b1, w2, b2, w3, b3, out_ref, ms_ref, cnt_ref,
             mx_ref):
    i = pl.program_id(0)
    t = dinv_ref[...] * (p_ref[0] + p_ref[1]) + cb[...]
    t = (t - bnm[...]) * lax.rsqrt(bnv[...] + 1e-5) * bng[...] + bnb[...]
    h = jnp.maximum(t, 0.0)

    @pl.when(i == 0)
    def _():
        ms_ref[...] = jnp.zeros_like(ms_ref)
        cnt_ref[...] = jnp.zeros_like(cnt_ref)
        mx_ref[...] = jnp.zeros_like(mx_ref)

    brow = brow_ref[0]
    gid = lax.broadcasted_iota(jnp.int32, (G, BR), 0)
    onehot = (gid == brow).astype(jnp.float32)
    ms_ref[...] += jnp.dot(onehot, h, preferred_element_type=jnp.float32)
    cnt_ref[...] += jnp.sum(onehot, axis=1, keepdims=True)

    bcol = bcol_ref[...]
    bmin = jnp.min(brow)
    bmax = jnp.max(brow)
    for g in range(G):
        @pl.when((g >= bmin) & (g <= bmax))
        def _():
            m = jnp.where(bcol == g, h, 0.0)
            mx_ref[g:g + 1, :] = jnp.maximum(mx_ref[g:g + 1, :],
                                             jnp.max(m, axis=0, keepdims=True))

    @pl.when(i == NBLK - 1)
    def _():
        mean = ms_ref[...] / jnp.maximum(cnt_ref[...], 1.0)
        gv = jnp.concatenate([mean, mx_ref[...]], axis=1)
        o = jnp.dot(gv, w1[...], preferred_element_type=jnp.float32) + b1[...]
        o = jnp.maximum(o, 0.0)
        o = jnp.dot(o, w2[...], preferred_element_type=jnp.float32) + b2[...]
        o = jnp.maximum(o, 0.0)
        out_ref[...] = jnp.dot(o, w3[...],
                               preferred_element_type=jnp.float32) + b3[...]


_k4 = pl.pallas_call(
    _k4_body,
    grid=(NBLK,),
    in_specs=[
        pl.BlockSpec((NC, BR, H), lambda i: (0, i, 0)),
        pl.BlockSpec((BR, 1), lambda i: (i, 0)),
        pl.BlockSpec((1, H), lambda i: (0, 0)),
        pl.BlockSpec((1, H), lambda i: (0, 0)),
        pl.BlockSpec((1, H), lambda i: (0, 0)),
        pl.BlockSpec((1, H), lambda i: (0, 0)),
        pl.BlockSpec((1, H), lambda i: (0, 0)),
        pl.BlockSpec((1, 1, BR), lambda i: (i, 0, 0)),
        pl.BlockSpec((BR, 1), lambda i: (i, 0)),
        pl.BlockSpec((2 * H, H), lambda i: (0, 0)),
        pl.BlockSpec((1, H), lambda i: (0, 0)),
        pl.BlockSpec((H, H // 2), lambda i: (0, 0)),
        pl.BlockSpec((1, H // 2), lambda i: (0, 0)),
        pl.BlockSpec((H // 2, 1), lambda i: (0, 0)),
        pl.BlockSpec((1, 1), lambda i: (0, 0)),
    ],
    out_specs=pl.BlockSpec((G, 1), lambda i: (0, 0)),
    out_shape=jax.ShapeDtypeStruct((G, 1), jnp.float32),
    scratch_shapes=[
        pltpu.VMEM((G, H), jnp.float32),
        pltpu.VMEM((G, 1), jnp.float32),
        pltpu.VMEM((G, H), jnp.float32),
    ],
)


def kernel(x, edge_index, batch, W_embed, b_embed, conv_W, conv_b, bn_gamma,
           bn_beta, bn_mean, bn_var, W1, b1, W2, b2, W3, b3):
    src = edge_index[0].reshape(NW, NSUP, 1, SB, CH)
    dst = edge_index[1].reshape(NW, NSUP, 1, SB, CH)
    idx = jnp.concatenate([src, dst], axis=2)
    degp = _deg_kernel(edge_index[1].reshape(NW, NCHUNK, CH))
    d0 = degp[0, :, 0].reshape(NPAD, 1)
    d1 = degp[1, :, 0].reshape(NPAD, 1)

    hs, dinv = _k1(x, W_embed, b_embed.reshape(1, H), d0, d1, conv_W[0])
    for i in range(L - 1):
        p = _msg_kernel(hs, idx)
        hs = _layer(p, dinv, bn_mean[i].reshape(1, H),
                    bn_var[i].reshape(1, H), bn_gamma[i].reshape(1, H),
                    bn_beta[i].reshape(1, H), conv_b[i].reshape(1, H),
                    conv_W[i + 1])
    p = _msg_kernel(hs, idx)
    i = L - 1
    return _k4(p, dinv, bn_mean[i].reshape(1, H),
               bn_var[i].reshape(1, H), bn_gamma[i].reshape(1, H),
               bn_beta[i].reshape(1, H), conv_b[i].reshape(1, H),
               batch.reshape(NBLK, 1, BR), batch.reshape(N, 1),
               W1, b1.reshape(1, H), W2, b2.reshape(1, H // 2),
               W3, b3.reshape(1, 1))


# speedup vs baseline: 1.0179x; 1.0179x over previous
import functools

import jax
import jax.numpy as jnp
from jax import lax
from jax.experimental import pallas as pl
from jax.experimental.pallas import tpu as pltpu
from jax.experimental.pallas import tpu_sc as plsc

N, E, H, G, L = 10000, 320000, 128, 64, 3
NC, NS = 2, 16
NW = NC * NS
EPW = E // NW
CH = 100
NCHUNK = EPW // CH
SB = 10
NSUP = NCHUNK // SB
NP2 = 10240
RPT = NP2 // NS
ZCH = 32
NPAD = 10240
DSL = NPAD // NS
BR = 1000
NBLK = N // BR


@functools.partial(
    pl.kernel,
    out_type=jax.ShapeDtypeStruct((NC, NPAD, 16), jnp.float32),
    mesh=plsc.VectorSubcoreMesh(core_axis_name="c", subcore_axis_name="s"),
    scratch_types=[
        pltpu.VMEM((NCHUNK, CH), jnp.int32),
        pltpu.VMEM((CH, 16), jnp.float32),
        pltpu.VMEM((128, 16), jnp.float32),
        pltpu.VMEM_SHARED((NPAD, 16), jnp.float32),
    ],
)
def _deg_kernel(dst_hbm, degp_hbm, idv, ones_v, zb, acc):
    cid = lax.axis_index("c")
    sid = lax.axis_index("s")
    wid = sid * NC + cid
    zeros16 = jnp.zeros((16,), jnp.float32)
    ones16 = jnp.ones((16,), jnp.float32)

    def fill(r, c):
        zb[r, :] = zeros16
        return c

    lax.fori_loop(0, 128, fill, 0)

    def fill1(r, c):
        ones_v[r, :] = ones16
        return c

    lax.fori_loop(0, CH, fill1, 0)
    for t in range(DSL // 128):
        pltpu.sync_copy(zb, acc.at[pl.ds(sid * DSL + t * 128, 128)])
    pltpu.sync_copy(dst_hbm.at[wid], idv)
    plsc.subcore_barrier()

    def cb(ch, c):
        pltpu.sync_copy(ones_v, acc.at[idv.at[ch]], add=True)
        return c

    lax.fori_loop(0, NCHUNK, cb, 0)
    plsc.subcore_barrier()
    pltpu.sync_copy(acc.at[pl.ds(sid * DSL, DSL)],
                    degp_hbm.at[cid, pl.ds(sid * DSL, DSL)])


@functools.partial(
    pl.kernel,
    out_type=jax.ShapeDtypeStruct((NC, NP2, H), jnp.float32),
    mesh=plsc.VectorSubcoreMesh(core_axis_name="c", subcore_axis_name="s"),
    scratch_types=[
        pltpu.VMEM((2, SB, CH), jnp.int32),
        pltpu.VMEM((CH, H), jnp.float32),
        pltpu.VMEM((CH, H), jnp.float32),
        pltpu.VMEM((CH, H), jnp.float32),
        pltpu.VMEM((ZCH, H), jnp.float32),
        pltpu.VMEM_SHARED((NP2, H), jnp.float32),
        pltpu.SemaphoreType.DMA,
        pltpu.SemaphoreType.DMA,
        pltpu.SemaphoreType.DMA,
        pltpu.SemaphoreType.DMA,
        pltpu.SemaphoreType.DMA,
        pltpu.SemaphoreType.DMA,
    ],
)
def _msg_kernel(hs_hbm, idx_hbm, p_hbm, ibuf, buf0, buf1, buf2, zb, acc,
                sg0, sg1, sg2, ss0, ss1, ss2):
    cid = lax.axis_index("c")
    sid = lax.axis_index("s")
    wid = sid * NC + cid
    zeros16 = jnp.zeros((16,), jnp.float32)

    @pl.when(cid == 0)
    def _():
        pltpu.sync_copy(hs_hbm.at[pl.ds(sid * RPT, RPT)],
                        acc.at[pl.ds(sid * RPT, RPT)])

    @pl.when(cid == 1)
    def _():
        def z1(r, c):
            for cc in range(H // 16):
                zb[r, pl.ds(cc * 16, 16)] = zeros16
            return c

        lax.fori_loop(0, ZCH, z1, 0)
        for t in range(RPT // ZCH):
            pltpu.sync_copy(zb, acc.at[pl.ds(sid * RPT + t * ZCH, ZCH)])

    plsc.subcore_barrier()

    bufs = (buf0, buf1, buf2)
    sgs = (sg0, sg1, sg2)
    sss = (ss0, ss1, ss2)

    def sup(s, c):
        pltpu.sync_copy(idx_hbm.at[wid, s], ibuf)
        pltpu.async_copy(hs_hbm.at[ibuf.at[0, 0]], bufs[0], sgs[0])
        pltpu.async_copy(hs_hbm.at[ibuf.at[0, 1]], bufs[1], sgs[1])
        for ch in range(SB):
            b = ch % 3
            pltpu.make_async_copy(hs_hbm.at[ibuf.at[0, ch]], bufs[b],
                                  sgs[b]).wait()
            pltpu.async_copy(bufs[b], acc.at[ibuf.at[1, ch]], sss[b],
                             add=True)
            if ch + 2 < SB:
                nb = (ch + 2) % 3
                if ch >= 1:
                    pltpu.make_async_copy(bufs[nb],
                                          acc.at[ibuf.at[1, ch - 1]],
                                          sss[nb]).wait()
                pltpu.async_copy(hs_hbm.at[ibuf.at[0, ch + 2]], bufs[nb],
                                 sgs[nb])
        for ch in range(SB - 3, SB):
            pltpu.make_async_copy(bufs[ch % 3], acc.at[ibuf.at[1, ch]],
                                  sss[ch % 3]).wait()
        return c

    lax.fori_loop(0, NSUP, sup, 0)
    plsc.subcore_barrier()
    pltpu.sync_copy(acc.at[pl.ds(sid * RPT, RPT)],
                    p_hbm.at[cid, pl.ds(sid * RPT, RPT)])


def _k1_body(x_ref, we_ref, be_ref, d0_ref, d1_ref, w0_ref, hs_ref, dinv_ref):
    deg = d0_ref[...] + d1_ref[...] + 1.0
    dinv = lax.rsqrt(deg)
    h0 = jnp.dot(x_ref[...], we_ref[...], preferred_element_type=jnp.float32)
    h0 = jnp.maximum(h0 + be_ref[...], 0.0)
    hs = jnp.dot(h0, w0_ref[...], preferred_element_type=jnp.float32) * dinv
    hs_ref[...] = hs
    dinv_ref[...] = dinv


_k1 = pl.pallas_call(
    _k1_body,
    grid=(NBLK,),
    in_specs=[
        pl.BlockSpec((BR, H), lambda i: (i, 0)),
        pl.BlockSpec((H, H), lambda i: (0, 0)),
        pl.BlockSpec((1, H), lambda i: (0, 0)),
        pl.BlockSpec((BR, 1), lambda i: (i, 0)),
        pl.BlockSpec((BR, 1), lambda i: (i, 0)),
        pl.BlockSpec((H, H), lambda i: (0, 0)),
    ],
    out_specs=[
        pl.BlockSpec((BR, H), lambda i: (i, 0)),
        pl.BlockSpec((BR, 1), lambda i: (i, 0)),
    ],
    out_shape=[
        jax.ShapeDtypeStruct((NP2, H), jnp.float32),
        jax.ShapeDtypeStruct((N, 1), jnp.float32),
    ],
)


def _layer_body(p_ref, dinv_ref, bnm, bnv, bng, bnb, cb, wn_ref,
                out_ref):
    dinv = dinv_ref[...]
    t = dinv * (p_ref[0] + p_ref[1]) + cb[...]
    t = (t - bnm[...]) * lax.rsqrt(bnv[...] + 1e-5) * bng[...] + bnb[...]
    h = jnp.maximum(t, 0.0)
    out_ref[...] = jnp.dot(h, wn_ref[...],
                           preferred_element_type=jnp.float32) * dinv


_layer = pl.pallas_call(
    _layer_body,
    grid=(NBLK,),
    in_specs=[
        pl.BlockSpec((NC, BR, H), lambda i: (0, i, 0)),
        pl.BlockSpec((BR, 1), lambda i: (i, 0)),
        pl.BlockSpec((1, H), lambda i: (0, 0)),
        pl.BlockSpec((1, H), lambda i: (0, 0)),
        pl.BlockSpec((1, H), lambda i: (0, 0)),
        pl.BlockSpec((1, H), lambda i: (0, 0)),
        pl.BlockSpec((1, H), lambda i: (0, 0)),
        pl.BlockSpec((H, H), lambda i: (0, 0)),
    ],
    out_specs=pl.BlockSpec((BR, H), lambda i: (i, 0)),
    out_shape=jax.ShapeDtypeStruct((NP2, H), jnp.float32),
)


def _k4_body(p_ref, dinv_ref, bnm, bnv, bng, bnb, cb, brow_ref,
             bcol_ref, w1, b1, w2, b2, w3, b3, out_ref, ms_ref, cnt_ref,
             mx_ref):
    i = pl.program_id(0)
    t = dinv_ref[...] * (p_ref[0] + p_ref[1]) + cb[...]
    t = (t - bnm[...]) * lax.rsqrt(bnv[...] + 1e-5) * bng[...] + bnb[...]
    h = jnp.maximum(t, 0.0)

    @pl.when(i == 0)
    def _():
        ms_ref[...] = jnp.zeros_like(ms_ref)
        cnt_ref[...] = jnp.zeros_like(cnt_ref)
        mx_ref[...] = jnp.zeros_like(mx_ref)

    brow = brow_ref[0]
    gid = lax.broadcasted_iota(jnp.int32, (G, BR), 0)
    onehot = (gid == brow).astype(jnp.float32)
    ms_ref[...] += jnp.dot(onehot, h, preferred_element_type=jnp.float32)
    cnt_ref[...] += jnp.sum(onehot, axis=1, keepdims=True)

    bcol = bcol_ref[...]
    bmin = jnp.min(brow)
    bmax = jnp.max(brow)
    for g in range(G):
        @pl.when((g >= bmin) & (g <= bmax))
        def _():
            m = jnp.where(bcol == g, h, 0.0)
            mx_ref[g:g + 1, :] = jnp.maximum(mx_ref[g:g + 1, :],
                                             jnp.max(m, axis=0, keepdims=True))

    @pl.when(i == NBLK - 1)
    def _():
        mean = ms_ref[...] / jnp.maximum(cnt_ref[...], 1.0)
        gv = jnp.concatenate([mean, mx_ref[...]], axis=1)
        o = jnp.dot(gv, w1[...], preferred_element_type=jnp.float32) + b1[...]
        o = jnp.maximum(o, 0.0)
        o = jnp.dot(o, w2[...], preferred_element_type=jnp.float32) + b2[...]
        o = jnp.maximum(o, 0.0)
        out_ref[...] = jnp.dot(o, w3[...],
                               preferred_element_type=jnp.float32) + b3[...]


_k4 = pl.pallas_call(
    _k4_body,
    grid=(NBLK,),
    in_specs=[
        pl.BlockSpec((NC, BR, H), lambda i: (0, i, 0)),
        pl.BlockSpec((BR, 1), lambda i: (i, 0)),
        pl.BlockSpec((1, H), lambda i: (0, 0)),
        pl.BlockSpec((1, H), lambda i: (0, 0)),
        pl.BlockSpec((1, H), lambda i: (0, 0)),
        pl.BlockSpec((1, H), lambda i: (0, 0)),
        pl.BlockSpec((1, H), lambda i: (0, 0)),
        pl.BlockSpec((1, 1, BR), lambda i: (i, 0, 0)),
        pl.BlockSpec((BR, 1), lambda i: (i, 0)),
        pl.BlockSpec((2 * H, H), lambda i: (0, 0)),
        pl.BlockSpec((1, H), lambda i: (0, 0)),
        pl.BlockSpec((H, H // 2), lambda i: (0, 0)),
        pl.BlockSpec((1, H // 2), lambda i: (0, 0)),
        pl.BlockSpec((H // 2, 1), lambda i: (0, 0)),
        pl.BlockSpec((1, 1), lambda i: (0, 0)),
    ],
    out_specs=pl.BlockSpec((G, 1), lambda i: (0, 0)),
    out_shape=jax.ShapeDtypeStruct((G, 1), jnp.float32),
    scratch_shapes=[
        pltpu.VMEM((G, H), jnp.float32),
        pltpu.VMEM((G, 1), jnp.float32),
        pltpu.VMEM((G, H), jnp.float32),
    ],
)


def kernel(x, edge_index, batch, W_embed, b_embed, conv_W, conv_b, bn_gamma,
           bn_beta, bn_mean, bn_var, W1, b1, W2, b2, W3, b3):
    src = edge_index[0].reshape(NW, NSUP, 1, SB, CH)
    dst = edge_index[1].reshape(NW, NSUP, 1, SB, CH)
    idx = jnp.concatenate([src, dst], axis=2)
    degp = _deg_kernel(edge_index[1].reshape(NW, NCHUNK, CH))
    d0 = degp[0, :, 0].reshape(NPAD, 1)
    d1 = degp[1, :, 0].reshape(NPAD, 1)

    hs, dinv = _k1(x, W_embed, b_embed.reshape(1, H), d0, d1, conv_W[0])
    for i in range(L - 1):
        p = _msg_kernel(hs, idx)
        hs = _layer(p, dinv, bn_mean[i].reshape(1, H),
                    bn_var[i].reshape(1, H), bn_gamma[i].reshape(1, H),
                    bn_beta[i].reshape(1, H), conv_b[i].reshape(1, H),
                    conv_W[i + 1])
    p = _msg_kernel(hs, idx)
    i = L - 1
    return _k4(p, dinv, bn_mean[i].reshape(1, H),
               bn_var[i].reshape(1, H), bn_gamma[i].reshape(1, H),
               bn_beta[i].reshape(1, H), conv_b[i].reshape(1, H),
               batch.reshape(NBLK, 1, BR), batch.reshape(N, 1),
               W1, b1.reshape(1, H), W2, b2.reshape(1, H // 2),
               W3, b3.reshape(1, 1))
